# baseline (speedup 1.0000x reference)
import functools

import jax
import jax.numpy as jnp
import numpy as np
from jax.experimental import pallas as pl
from jax.experimental.pallas import tpu as pltpu
from jax.sharding import Mesh, PartitionSpec as P

try:
    from jax.experimental.shard_map import shard_map as _shard_map
except ImportError:
    _shard_map = jax.shard_map

N = 10000
NFEAT = 128
NHID = 16
NCLASS = 7


def _p0_body(br, adj_ref, x_ref, w1_ref, b1_ref, w2_ref, o_ref, s1_scr):
    g = pl.program_id(0)

    @pl.when(g == 0)
    def _():
        s1_scr[...] = jnp.dot(x_ref[...], w1_ref[...],
                              preferred_element_type=jnp.float32)

    h = jnp.dot(adj_ref[0], s1_scr[...], preferred_element_type=jnp.float32)
    h = jnp.maximum(h + b1_ref[...], 0.0)
    o_ref[pl.ds(g * br, br), :] = jnp.dot(h, w2_ref[...],
                                          preferred_element_type=jnp.float32)


def _p1_body(br, adj_ref, s2_ref, b2_ref, wl_ref, bl_ref, o_ref):
    g = pl.program_id(0)
    h2 = jnp.dot(adj_ref[0], s2_ref[...],
                 preferred_element_type=jnp.float32) + b2_ref[...]
    o = jnp.dot(h2, wl_ref[...], preferred_element_type=jnp.float32) + bl_ref[...]
    m = jnp.max(o, axis=-1, keepdims=True)
    e = o - m
    o_ref[pl.ds(g * br, br), :] = e - jnp.log(
        jnp.sum(jnp.exp(e), axis=-1, keepdims=True))


def _shard_fwd(x, adj, W1, b1r, W2, b2r, WL, bLr):
    nl = adj.shape[1]
    br = 200 if nl % 400 else 400
    nb = nl // br
    c = lambda g: (0, 0)

    s2_loc = pl.pallas_call(
        functools.partial(_p0_body, br),
        grid=(nb,),
        in_specs=[
            pl.BlockSpec((1, br, N), lambda g: (0, g, 0)),
            pl.BlockSpec((N, NFEAT), c),
            pl.BlockSpec((NFEAT, NHID), c),
            pl.BlockSpec((1, NHID), c),
            pl.BlockSpec((NHID, NCLASS), c),
        ],
        out_specs=pl.BlockSpec((nl, NCLASS), c),
        out_shape=jax.ShapeDtypeStruct((nl, NCLASS), jnp.float32),
        scratch_shapes=[pltpu.VMEM((N, NHID), jnp.float32)],
    )(adj, x, W1, b1r, W2)

    s2_full = jax.lax.all_gather(s2_loc, "d", axis=0, tiled=True)

    return pl.pallas_call(
        functools.partial(_p1_body, br),
        grid=(nb,),
        in_specs=[
            pl.BlockSpec((1, br, N), lambda g: (1, g, 0)),
            pl.BlockSpec((N, NCLASS), c),
            pl.BlockSpec((1, NCLASS), c),
            pl.BlockSpec((NCLASS, NCLASS), c),
            pl.BlockSpec((1, NCLASS), c),
        ],
        out_specs=pl.BlockSpec((nl, NCLASS), c),
        out_shape=jax.ShapeDtypeStruct((nl, NCLASS), jnp.float32),
    )(adj, s2_full, b2r, WL, bLr)


def kernel(x, adj, W1, b1, W2, b2, WL, bL):
    b1r = b1.reshape(1, NHID)
    b2r = b2.reshape(1, NCLASS)
    bLr = bL.reshape(1, NCLASS)
    devs = jax.devices()
    nd = 2 if len(devs) >= 2 else 1
    mesh = Mesh(np.array(devs[:nd]), ("d",))
    kw = dict(
        mesh=mesh,
        in_specs=(P(), P(None, "d", None), P(), P(), P(), P(), P(), P()),
        out_specs=P("d", None),
    )
    try:
        f = _shard_map(_shard_fwd, check_rep=False, **kw)
    except TypeError:
        f = _shard_map(_shard_fwd, check_vma=False, **kw)
    return f(x, adj, W1, b1r, W2, b2r, WL, bLr)


# speedup vs baseline: 4.6867x; 4.6867x over previous
import jax
import jax.numpy as jnp
from jax.experimental import pallas as pl
from jax.experimental.pallas import tpu as pltpu

N = 10000
NFEAT = 128
NHID = 16
NCLASS = 7
BR = 400
NB = N // BR


def _body(adj_ref, x_ref, w1_ref, b1_ref, w2_ref, b2_ref, wl_ref, bl_ref,
          out_ref, s1_scr, s2_scr):
    g = pl.program_id(0)
    i = jax.lax.rem(g, NB)

    @pl.when(g == 0)
    def _():
        s1_scr[...] = jnp.dot(x_ref[...], w1_ref[...],
                              preferred_element_type=jnp.float32)

    @pl.when(g < NB)
    def _():
        h = jnp.dot(adj_ref[0], s1_scr[...],
                    preferred_element_type=jnp.float32)
        h = jnp.maximum(h + b1_ref[...], 0.0)
        s2_scr[pl.ds(i * BR, BR), :] = jnp.dot(
            h, w2_ref[...], preferred_element_type=jnp.float32)

    @pl.when(g >= NB)
    def _():
        h2 = jnp.dot(adj_ref[0], s2_scr[...],
                     preferred_element_type=jnp.float32) + b2_ref[...]
        o = jnp.dot(h2, wl_ref[...],
                    preferred_element_type=jnp.float32) + bl_ref[...]
        m = jnp.max(o, axis=-1, keepdims=True)
        e = o - m
        out_ref[pl.ds(i * BR, BR), :] = e - jnp.log(
            jnp.sum(jnp.exp(e), axis=-1, keepdims=True))


def kernel(x, adj, W1, b1, W2, b2, WL, bL):
    b1r = b1.reshape(1, NHID)
    b2r = b2.reshape(1, NCLASS)
    bLr = bL.reshape(1, NCLASS)
    c = lambda g: (0, 0)
    return pl.pallas_call(
        _body,
        grid=(2 * NB,),
        in_specs=[
            pl.BlockSpec((1, BR, N), lambda g: (g // NB, g % NB, 0)),
            pl.BlockSpec((N, NFEAT), c),
            pl.BlockSpec((NFEAT, NHID), c),
            pl.BlockSpec((1, NHID), c),
            pl.BlockSpec((NHID, NCLASS), c),
            pl.BlockSpec((1, NCLASS), c),
            pl.BlockSpec((NCLASS, NCLASS), c),
            pl.BlockSpec((1, NCLASS), c),
        ],
        out_specs=pl.BlockSpec((N, NCLASS), c),
        out_shape=jax.ShapeDtypeStruct((N, NCLASS), jnp.float32),
        scratch_shapes=[
            pltpu.VMEM((N, NHID), jnp.float32),
            pltpu.VMEM((N, NCLASS), jnp.float32),
        ],
    )(adj, x, W1, b1r, W2, b2r, WL, bLr)
